# Initial kernel scaffold; baseline (speedup 1.0000x reference)
import functools

import jax
import jax.numpy as jnp
from jax import lax
from jax.experimental import pallas as pl
from jax.experimental.pallas import tpu as pltpu
from jax.experimental.pallas import tpu_sc as plsc

N_NODES = 10000
N_EDGES = 160000
NODE_DIM = 256
MAX_DEG = 512

NC = 2
NS = 16
NW = NC * NS
L = 16

N_EP = 2 * N_EDGES
EP_PER_TILE = N_EP // NW
HIST_PAD = 10240
COLS_PER_TILE = HIST_PAD // NS

CHUNK = 80
N_CHUNKS = N_NODES // CHUNK
MAX_CHUNKS_PER_TILE = -(-N_CHUNKS // NW)

_MESH = plsc.VectorSubcoreMesh(
    core_axis_name="c", subcore_axis_name="s", num_cores=NC, num_subcores=NS
)


def _hist_body(edges_hbm, partial_hbm, ep_v, hist_v, colbuf_v, red_v, shared_hist):
    cid = lax.axis_index("c")
    sid = lax.axis_index("s")
    wid = cid * NS + sid

    pltpu.sync_copy(edges_hbm.at[pl.ds(wid * EP_PER_TILE, EP_PER_TILE)], ep_v)

    zeros = jnp.zeros((L,), jnp.int32)

    def zero_step(j, _):
        hist_v[pl.ds(j * L, L)] = zeros
        return 0

    lax.fori_loop(0, HIST_PAD // L, zero_step, 0)

    def hist_step(j, _):
        v = ep_v[pl.ds(j * L, L)]
        cnt, last = plsc.scan_count(v)
        plsc.addupdate_scatter(hist_v, [v], cnt, mask=last)
        return 0

    lax.fori_loop(0, EP_PER_TILE // L, hist_step, 0)

    pltpu.sync_copy(hist_v, shared_hist.at[sid])
    plsc.subcore_barrier()
    pltpu.sync_copy(shared_hist.at[:, pl.ds(sid * COLS_PER_TILE, COLS_PER_TILE)],
                    colbuf_v)

    def red_step(j, _):
        acc = colbuf_v[0, pl.ds(j * L, L)]
        for r in range(1, NS):
            acc = acc + colbuf_v[r, pl.ds(j * L, L)]
        red_v[pl.ds(j * L, L)] = acc
        return 0

    lax.fori_loop(0, COLS_PER_TILE // L, red_step, 0)

    pltpu.sync_copy(red_v, partial_hbm.at[cid, pl.ds(sid * COLS_PER_TILE,
                                                     COLS_PER_TILE)])


_hist_kernel = functools.partial(
    pl.kernel,
    out_type=jax.ShapeDtypeStruct((NC, HIST_PAD), jnp.int32),
    mesh=_MESH,
    scratch_types=[
        pltpu.VMEM((EP_PER_TILE,), jnp.int32),
        pltpu.VMEM((HIST_PAD,), jnp.int32),
        pltpu.VMEM((NS, COLS_PER_TILE), jnp.int32),
        pltpu.VMEM((COLS_PER_TILE,), jnp.int32),
        pltpu.VMEM_SHARED((NS, HIST_PAD), jnp.int32),
    ],
)(_hist_body)


def _gather_body(x_hbm, partial_hbm, zsum_hbm, out_hbm,
                 deg2_v, idx_v, xbuf_v, zbuf_v, sem):
    cid = lax.axis_index("c")
    sid = lax.axis_index("s")
    wid = cid * NS + sid

    for t in range(MAX_CHUNKS_PER_TILE):
        chunk = wid + NW * t

        @pl.when(chunk < N_CHUNKS)
        def _():
            base = chunk * CHUNK
            pltpu.sync_copy(partial_hbm.at[:, pl.ds(base, CHUNK)], deg2_v)
            for j in range(CHUNK // L):
                p0 = deg2_v[0, pl.ds(j * L, L)]
                p1 = deg2_v[1, pl.ds(j * L, L)]
                idx_v[pl.ds(j * L, L)] = jnp.minimum(p0 + p1, MAX_DEG - 1)
            cp = pltpu.async_copy(zsum_hbm.at[idx_v], zbuf_v, sem)
            pltpu.sync_copy(x_hbm.at[pl.ds(base, CHUNK)], xbuf_v)
            cp.wait()

            def add_step(r, _):
                for cj in range(NODE_DIM // L):
                    xbuf_v[r, pl.ds(cj * L, L)] = (
                        xbuf_v[r, pl.ds(cj * L, L)] + zbuf_v[r, pl.ds(cj * L, L)]
                    )
                return 0

            lax.fori_loop(0, CHUNK, add_step, 0)
            pltpu.sync_copy(xbuf_v, out_hbm.at[pl.ds(base, CHUNK)])


_gather_kernel = functools.partial(
    pl.kernel,
    out_type=jax.ShapeDtypeStruct((N_NODES, NODE_DIM), jnp.float32),
    mesh=_MESH,
    scratch_types=[
        pltpu.VMEM((NC, CHUNK), jnp.int32),
        pltpu.VMEM((CHUNK,), jnp.int32),
        pltpu.VMEM((CHUNK, NODE_DIM), jnp.float32),
        pltpu.VMEM((CHUNK, NODE_DIM), jnp.float32),
        pltpu.SemaphoreType.DMA,
    ],
)(_gather_body)


def _zsum_body(zin_ref, zout_ref, o_ref):
    o_ref[...] = zin_ref[...] + zout_ref[...]


def _zsum_tc(z_in, z_out):
    return pl.pallas_call(
        _zsum_body,
        out_shape=jax.ShapeDtypeStruct((MAX_DEG, NODE_DIM), jnp.float32),
    )(z_in, z_out)


def kernel(x, edge_index, z_in, z_out):
    edges = edge_index.reshape(-1)
    partial = _hist_kernel(edges)
    zsum = _zsum_tc(z_in, z_out)
    return _gather_kernel(x, partial, zsum)


# speedup vs baseline: 1.3533x; 1.3533x over previous
import functools

import jax
import jax.numpy as jnp
from jax import lax
from jax.experimental import pallas as pl
from jax.experimental.pallas import tpu as pltpu
from jax.experimental.pallas import tpu_sc as plsc

N_NODES = 10000
N_EDGES = 160000
NODE_DIM = 256
MAX_DEG = 512

NC = 2
NS = 16
NW = NC * NS
L = 16

N_EP = 2 * N_EDGES
EP_PER_TILE = N_EP // NW
HIST_PAD = 10240
COLS_PER_TILE = HIST_PAD // NS

CHUNK = 80
N_CHUNKS = N_NODES // CHUNK
MAX_CHUNKS_PER_TILE = -(-N_CHUNKS // NW)

_MESH = plsc.VectorSubcoreMesh(
    core_axis_name="c", subcore_axis_name="s", num_cores=NC, num_subcores=NS
)


def _hist_body(edges_hbm, partial_hbm, ep_v, hist_v, colbuf_v, red_v, shared_hist):
    cid = lax.axis_index("c")
    sid = lax.axis_index("s")
    wid = cid * NS + sid

    pltpu.sync_copy(edges_hbm.at[pl.ds(wid * EP_PER_TILE, EP_PER_TILE)], ep_v)

    zeros = jnp.zeros((L,), jnp.int32)

    def zero_step(j, _):
        hist_v[pl.ds(j * L, L)] = zeros
        return 0

    lax.fori_loop(0, HIST_PAD // L, zero_step, 0)

    def hist_step(j, _):
        v = ep_v[pl.ds(j * L, L)]
        cnt, last = plsc.scan_count(v)
        plsc.addupdate_scatter(hist_v, [v], cnt, mask=last)
        return 0

    lax.fori_loop(0, EP_PER_TILE // L, hist_step, 0)

    pltpu.sync_copy(hist_v, shared_hist.at[sid])
    plsc.subcore_barrier()
    pltpu.sync_copy(shared_hist.at[:, pl.ds(sid * COLS_PER_TILE, COLS_PER_TILE)],
                    colbuf_v)

    def red_step(j, _):
        acc = colbuf_v[0, pl.ds(j * L, L)]
        for r in range(1, NS):
            acc = acc + colbuf_v[r, pl.ds(j * L, L)]
        red_v[pl.ds(j * L, L)] = acc
        return 0

    lax.fori_loop(0, COLS_PER_TILE // L, red_step, 0)

    pltpu.sync_copy(
        red_v,
        partial_hbm.at[pl.ds(cid * HIST_PAD + sid * COLS_PER_TILE,
                             COLS_PER_TILE)],
    )


_hist_kernel = functools.partial(
    pl.kernel,
    out_type=jax.ShapeDtypeStruct((NC * HIST_PAD,), jnp.int32),
    mesh=_MESH,
    compiler_params=pltpu.CompilerParams(needs_layout_passes=False),
    scratch_types=[
        pltpu.VMEM((EP_PER_TILE,), jnp.int32),
        pltpu.VMEM((HIST_PAD,), jnp.int32),
        pltpu.VMEM((NS, COLS_PER_TILE), jnp.int32),
        pltpu.VMEM((COLS_PER_TILE,), jnp.int32),
        pltpu.VMEM_SHARED((NS, HIST_PAD), jnp.int32),
    ],
)(_hist_body)


def _gather_body(x_hbm, partial_hbm, zsum_hbm, out_hbm,
                 p0_v, p1_v, idx_v, xbuf_v, zbuf_v, sem):
    cid = lax.axis_index("c")
    sid = lax.axis_index("s")
    wid = cid * NS + sid

    for t in range(MAX_CHUNKS_PER_TILE):
        chunk = wid + NW * t

        @pl.when(chunk < N_CHUNKS)
        def _():
            base = chunk * CHUNK
            pltpu.sync_copy(partial_hbm.at[pl.ds(base, CHUNK)], p0_v)
            pltpu.sync_copy(partial_hbm.at[pl.ds(HIST_PAD + base, CHUNK)], p1_v)
            for j in range(CHUNK // L):
                p0 = p0_v[pl.ds(j * L, L)]
                p1 = p1_v[pl.ds(j * L, L)]
                idx_v[pl.ds(j * L, L)] = jnp.minimum(p0 + p1, MAX_DEG - 1)
            cp = pltpu.async_copy(zsum_hbm.at[idx_v], zbuf_v, sem)
            pltpu.sync_copy(x_hbm.at[pl.ds(base, CHUNK)], xbuf_v)
            cp.wait()

            def add_step(r, _):
                for cj in range(NODE_DIM // L):
                    xbuf_v[r, pl.ds(cj * L, L)] = (
                        xbuf_v[r, pl.ds(cj * L, L)] + zbuf_v[r, pl.ds(cj * L, L)]
                    )
                return 0

            lax.fori_loop(0, CHUNK, add_step, 0)
            pltpu.sync_copy(xbuf_v, out_hbm.at[pl.ds(base, CHUNK)])


_gather_kernel = functools.partial(
    pl.kernel,
    out_type=jax.ShapeDtypeStruct((N_NODES, NODE_DIM), jnp.float32),
    mesh=_MESH,
    scratch_types=[
        pltpu.VMEM((CHUNK,), jnp.int32),
        pltpu.VMEM((CHUNK,), jnp.int32),
        pltpu.VMEM((CHUNK,), jnp.int32),
        pltpu.VMEM((CHUNK, NODE_DIM), jnp.float32),
        pltpu.VMEM((CHUNK, NODE_DIM), jnp.float32),
        pltpu.SemaphoreType.DMA,
    ],
)(_gather_body)


def _zsum_body(zin_ref, zout_ref, o_ref):
    o_ref[...] = zin_ref[...] + zout_ref[...]


def _zsum_tc(z_in, z_out):
    return pl.pallas_call(
        _zsum_body,
        out_shape=jax.ShapeDtypeStruct((MAX_DEG, NODE_DIM), jnp.float32),
    )(z_in, z_out)


def kernel(x, edge_index, z_in, z_out):
    edges = edge_index.reshape(-1)
    partial = _hist_kernel(edges)
    zsum = _zsum_tc(z_in, z_out)
    return _gather_kernel(x, partial, zsum)


# speedup vs baseline: 1.4388x; 1.0631x over previous
import functools

import jax
import jax.numpy as jnp
from jax import lax
from jax.experimental import pallas as pl
from jax.experimental.pallas import tpu as pltpu
from jax.experimental.pallas import tpu_sc as plsc

N_NODES = 10000
N_EDGES = 160000
NODE_DIM = 256
MAX_DEG = 512

NC = 2
NS = 16
NW = NC * NS
L = 16

N_EP = 2 * N_EDGES
EP_PER_TILE = N_EP // NW
HIST_PAD = 10240
COLS_PER_TILE = HIST_PAD // NS
Z_ROWS_PER_TILE = MAX_DEG // NW

CHUNK = 80
N_CHUNKS = N_NODES // CHUNK
MAX_T = -(-N_CHUNKS // NW)

_MESH = plsc.VectorSubcoreMesh(
    core_axis_name="c", subcore_axis_name="s", num_cores=NC, num_subcores=NS
)


def _hist_body(edges_hbm, zin_hbm, zout_hbm, partial_hbm, zsum_hbm,
               ep_v, hist_v, colbuf_v, red_v, zi_v, zo_v, shared_hist,
               sem_e, sem_z, sem_zo):
    cid = lax.axis_index("c")
    sid = lax.axis_index("s")
    wid = cid * NS + sid

    cp_ep = pltpu.async_copy(
        edges_hbm.at[pl.ds(wid * EP_PER_TILE, EP_PER_TILE)], ep_v, sem_e)
    zrow = wid * Z_ROWS_PER_TILE
    cp_zi = pltpu.async_copy(zin_hbm.at[pl.ds(zrow, Z_ROWS_PER_TILE)], zi_v, sem_z)
    cp_zo = pltpu.async_copy(zout_hbm.at[pl.ds(zrow, Z_ROWS_PER_TILE)], zo_v, sem_z)

    zeros = jnp.zeros((L,), jnp.int32)

    def zero_step(j, _):
        for u in range(4):
            hist_v[pl.ds((j * 4 + u) * L, L)] = zeros
        return 0

    lax.fori_loop(0, HIST_PAD // L // 4, zero_step, 0)

    cp_zi.wait()
    cp_zo.wait()

    def zsum_step(r, _):
        for cj in range(NODE_DIM // L):
            sl = pl.ds(cj * L, L)
            zi_v[r, sl] = zi_v[r, sl] + zo_v[r, sl]
        return 0

    lax.fori_loop(0, Z_ROWS_PER_TILE, zsum_step, 0)
    cp_zs = pltpu.async_copy(
        zi_v, zsum_hbm.at[pl.ds(zrow, Z_ROWS_PER_TILE)], sem_zo)

    cp_ep.wait()

    def hist_step(j, _):
        for u in range(5):
            v = ep_v[pl.ds((j * 5 + u) * L, L)]
            cnt, last = plsc.scan_count(v)
            plsc.addupdate_scatter(hist_v, [v], cnt, mask=last)
        return 0

    lax.fori_loop(0, EP_PER_TILE // L // 5, hist_step, 0)

    pltpu.sync_copy(hist_v, shared_hist.at[sid])
    plsc.subcore_barrier()
    pltpu.sync_copy(shared_hist.at[:, pl.ds(sid * COLS_PER_TILE, COLS_PER_TILE)],
                    colbuf_v)

    def red_step(j, _):
        acc = colbuf_v[0, pl.ds(j * L, L)]
        for r in range(1, NS):
            acc = acc + colbuf_v[r, pl.ds(j * L, L)]
        red_v[pl.ds(j * L, L)] = acc
        return 0

    lax.fori_loop(0, COLS_PER_TILE // L, red_step, 0)

    pltpu.sync_copy(
        red_v,
        partial_hbm.at[pl.ds(cid * HIST_PAD + sid * COLS_PER_TILE,
                             COLS_PER_TILE)],
    )
    cp_zs.wait()


_hist_kernel = functools.partial(
    pl.kernel,
    out_type=(
        jax.ShapeDtypeStruct((NC * HIST_PAD,), jnp.int32),
        jax.ShapeDtypeStruct((MAX_DEG, NODE_DIM), jnp.float32),
    ),
    mesh=_MESH,
    compiler_params=pltpu.CompilerParams(needs_layout_passes=False),
    scratch_types=[
        pltpu.VMEM((EP_PER_TILE,), jnp.int32),
        pltpu.VMEM((HIST_PAD,), jnp.int32),
        pltpu.VMEM((NS, COLS_PER_TILE), jnp.int32),
        pltpu.VMEM((COLS_PER_TILE,), jnp.int32),
        pltpu.VMEM((Z_ROWS_PER_TILE, NODE_DIM), jnp.float32),
        pltpu.VMEM((Z_ROWS_PER_TILE, NODE_DIM), jnp.float32),
        pltpu.VMEM_SHARED((NS, HIST_PAD), jnp.int32),
        pltpu.SemaphoreType.DMA,
        pltpu.SemaphoreType.DMA,
        pltpu.SemaphoreType.DMA,
    ],
)(_hist_body)


P_WORDS = MAX_T * CHUNK


def _gather_body(x_hbm, partial_hbm, zsum_hbm, out_hbm,
                 pa_v, pb_v, idx_v, x0_v, x1_v, x2_v, z0_v, z1_v,
                 sem_p, sem_x, sem_g, sem_o):
    cid = lax.axis_index("c")
    sid = lax.axis_index("s")
    wid = cid * NS + sid

    xbufs = [x0_v, x1_v, x2_v]
    zbufs = [z0_v, z1_v]

    def valid(t):
        return (MAX_T * wid + t) < N_CHUNKS

    def chunkid(t):
        return MAX_T * wid + t

    def x_desc(t):
        return pltpu.make_async_copy(
            x_hbm.at[chunkid(t)], xbufs[t % 3], sem_x.at[t])

    def g_desc(t):
        return pltpu.make_async_copy(
            zsum_hbm.at[idx_v.at[pl.ds(t * CHUNK, CHUNK)]], zbufs[t % 2],
            sem_g.at[t])

    def o_desc(t):
        return pltpu.make_async_copy(
            xbufs[t % 3], out_hbm.at[chunkid(t)], sem_o)

    def fire_x(t):
        pltpu.async_copy(x_hbm.at[chunkid(t)], xbufs[t % 3], sem_x.at[t])

    def fire_gather(t):
        pltpu.async_copy(
            zsum_hbm.at[idx_v.at[pl.ds(t * CHUNK, CHUNK)]], zbufs[t % 2],
            sem_g.at[t])

    def stage_add(t):
        x_desc(t).wait()
        g_desc(t).wait()
        xb, zb = xbufs[t % 3], zbufs[t % 2]

        def add_step(r, _):
            for cj in range(NODE_DIM // L):
                sl = pl.ds(cj * L, L)
                xb[r, sl] = xb[r, sl] + zb[r, sl]
            return 0

        lax.fori_loop(0, CHUNK, add_step, 0)
        pltpu.async_copy(xb, out_hbm.at[chunkid(t)], sem_o)

    dpa = pltpu.async_copy(
        partial_hbm.at[pl.ds(wid * P_WORDS, P_WORDS)], pa_v, sem_p)
    dpb = pltpu.async_copy(
        partial_hbm.at[pl.ds(HIST_PAD + wid * P_WORDS, P_WORDS)], pb_v, sem_p)
    pl.when(valid(0))(lambda: fire_x(0))
    pl.when(valid(1))(lambda: fire_x(1))
    dpa.wait()
    dpb.wait()
    for j in range(P_WORDS // L):
        sl = pl.ds(j * L, L)
        idx_v[sl] = jnp.minimum(pa_v[sl] + pb_v[sl], MAX_DEG - 1)
    pl.when(valid(0))(lambda: fire_gather(0))

    for t in range(MAX_T):
        if t + 1 < MAX_T:
            pl.when(valid(t + 1))(lambda t=t: fire_gather(t + 1))
        if t + 2 < MAX_T:
            def prefetch(t=t):
                if t - 1 >= 0:
                    o_desc(t - 1).wait()
                fire_x(t + 2)
            pl.when(valid(t + 2))(prefetch)
        pl.when(valid(t))(lambda t=t: stage_add(t))

    for t in range(MAX_T):
        if t + 3 >= MAX_T:
            pl.when(valid(t))(lambda t=t: o_desc(t).wait())


_gather_kernel = functools.partial(
    pl.kernel,
    out_type=jax.ShapeDtypeStruct((N_CHUNKS, CHUNK, NODE_DIM), jnp.float32),
    mesh=_MESH,
    scratch_types=[
        pltpu.VMEM((P_WORDS,), jnp.int32),
        pltpu.VMEM((P_WORDS,), jnp.int32),
        pltpu.VMEM((P_WORDS,), jnp.int32),
        pltpu.VMEM((CHUNK, NODE_DIM), jnp.float32),
        pltpu.VMEM((CHUNK, NODE_DIM), jnp.float32),
        pltpu.VMEM((CHUNK, NODE_DIM), jnp.float32),
        pltpu.VMEM((CHUNK, NODE_DIM), jnp.float32),
        pltpu.VMEM((CHUNK, NODE_DIM), jnp.float32),
        pltpu.SemaphoreType.DMA,
        pltpu.SemaphoreType.DMA((MAX_T,)),
        pltpu.SemaphoreType.DMA((MAX_T,)),
        pltpu.SemaphoreType.DMA,
    ],
)(_gather_body)


def kernel(x, edge_index, z_in, z_out):
    edges = edge_index.reshape(-1)
    partial, zsum = _hist_kernel(edges, z_in, z_out)
    x3 = x.reshape(N_CHUNKS, CHUNK, NODE_DIM)
    out3 = _gather_kernel(x3, partial, zsum)
    return out3.reshape(N_NODES, NODE_DIM)


# speedup vs baseline: 1.4926x; 1.0374x over previous
import functools

import jax
import jax.numpy as jnp
from jax import lax
from jax.experimental import pallas as pl
from jax.experimental.pallas import tpu as pltpu
from jax.experimental.pallas import tpu_sc as plsc

N_NODES = 10000
N_EDGES = 160000
NODE_DIM = 256
MAX_DEG = 512

NC = 2
NS = 16
NW = NC * NS
L = 16

N_EP = 2 * N_EDGES
EP_PER_TILE = N_EP // NW
HIST_PAD = 10240
COLS_PER_TILE = HIST_PAD // NS
Z_ROWS_PER_TILE = MAX_DEG // NW

CHUNK = 80
N_CHUNKS = N_NODES // CHUNK
MAX_T = -(-N_CHUNKS // NW)

_MESH = plsc.VectorSubcoreMesh(
    core_axis_name="c", subcore_axis_name="s", num_cores=NC, num_subcores=NS
)


def _hist_body(edges_hbm, zin_hbm, zout_hbm, partial_hbm, zsum_hbm,
               ep_v, hist_v, colbuf_v, red_v, zi_v, zo_v, shared_hist,
               sem_e, sem_z, sem_zo):
    cid = lax.axis_index("c")
    sid = lax.axis_index("s")
    wid = cid * NS + sid

    cp_ep = pltpu.async_copy(
        edges_hbm.at[pl.ds(wid * EP_PER_TILE, EP_PER_TILE)], ep_v, sem_e)
    zrow = wid * Z_ROWS_PER_TILE
    cp_zi = pltpu.async_copy(zin_hbm.at[pl.ds(zrow, Z_ROWS_PER_TILE)], zi_v, sem_z)
    cp_zo = pltpu.async_copy(zout_hbm.at[pl.ds(zrow, Z_ROWS_PER_TILE)], zo_v, sem_z)

    zeros = jnp.zeros((L,), jnp.int32)

    def zero_step(j, _):
        for u in range(4):
            hist_v[pl.ds((j * 4 + u) * L, L)] = zeros
        return 0

    lax.fori_loop(0, HIST_PAD // L // 4, zero_step, 0)

    cp_zi.wait()
    cp_zo.wait()

    def zsum_step(r, _):
        for cj in range(NODE_DIM // L):
            sl = pl.ds(cj * L, L)
            zi_v[r, sl] = zi_v[r, sl] + zo_v[r, sl]
        return 0

    lax.fori_loop(0, Z_ROWS_PER_TILE, zsum_step, 0)
    cp_zs = pltpu.async_copy(
        zi_v, zsum_hbm.at[pl.ds(zrow, Z_ROWS_PER_TILE)], sem_zo)

    cp_ep.wait()

    def hist_step(j, _):
        for u in range(5):
            v = ep_v[pl.ds((j * 5 + u) * L, L)]
            cnt, last = plsc.scan_count(v)
            plsc.addupdate_scatter(hist_v, [v], cnt, mask=last)
        return 0

    lax.fori_loop(0, EP_PER_TILE // L // 5, hist_step, 0)

    pltpu.sync_copy(hist_v, shared_hist.at[sid])
    plsc.subcore_barrier()
    pltpu.sync_copy(shared_hist.at[:, pl.ds(sid * COLS_PER_TILE, COLS_PER_TILE)],
                    colbuf_v)

    def red_step(j, _):
        acc = colbuf_v[0, pl.ds(j * L, L)]
        for r in range(1, NS):
            acc = acc + colbuf_v[r, pl.ds(j * L, L)]
        red_v[pl.ds(j * L, L)] = acc
        return 0

    lax.fori_loop(0, COLS_PER_TILE // L, red_step, 0)

    pltpu.sync_copy(
        red_v,
        partial_hbm.at[pl.ds(cid * HIST_PAD + sid * COLS_PER_TILE,
                             COLS_PER_TILE)],
    )
    cp_zs.wait()


_hist_kernel = functools.partial(
    pl.kernel,
    out_type=(
        jax.ShapeDtypeStruct((NC * HIST_PAD,), jnp.int32),
        jax.ShapeDtypeStruct((MAX_DEG, NODE_DIM), jnp.float32),
    ),
    mesh=_MESH,
    compiler_params=pltpu.CompilerParams(needs_layout_passes=False),
    scratch_types=[
        pltpu.VMEM((EP_PER_TILE,), jnp.int32),
        pltpu.VMEM((HIST_PAD,), jnp.int32),
        pltpu.VMEM((NS, COLS_PER_TILE), jnp.int32),
        pltpu.VMEM((COLS_PER_TILE,), jnp.int32),
        pltpu.VMEM((Z_ROWS_PER_TILE, NODE_DIM), jnp.float32),
        pltpu.VMEM((Z_ROWS_PER_TILE, NODE_DIM), jnp.float32),
        pltpu.VMEM_SHARED((NS, HIST_PAD), jnp.int32),
        pltpu.SemaphoreType.DMA,
        pltpu.SemaphoreType.DMA,
        pltpu.SemaphoreType.DMA,
    ],
)(_hist_body)


P_WORDS = MAX_T * CHUNK


HALF = NODE_DIM // 2
N_ITEMS = 2 * MAX_T


def _gather_body(x_hbm, partial_hbm, zsum_hbm, out_hbm,
                 pa_v, pb_v, idx_v, x0_v, x1_v, x2_v, tab_v, zs_sh,
                 sem_p, sem_x, sem_o):
    cid = lax.axis_index("c")
    sid = lax.axis_index("s")
    wid = cid * NS + sid

    xbufs = [x0_v, x1_v, x2_v]

    def tch(i):
        return i // MAX_T, i % MAX_T

    def valid(i):
        return (MAX_T * wid + tch(i)[1]) < N_CHUNKS

    def chunkid(i):
        return MAX_T * wid + tch(i)[1]

    def xslice(i):
        h = tch(i)[0]
        return pl.ds(h * HALF, HALF)

    def x_desc(i):
        return pltpu.make_async_copy(
            x_hbm.at[chunkid(i), :, xslice(i)], xbufs[i % 3], sem_x.at[i])

    def o_desc(i):
        return pltpu.make_async_copy(
            xbufs[i % 3], out_hbm.at[chunkid(i), :, xslice(i)], sem_o)

    def fire_x(i):
        pltpu.async_copy(
            x_hbm.at[chunkid(i), :, xslice(i)], xbufs[i % 3], sem_x.at[i])

    def load_table(h):
        pltpu.sync_copy(zs_sh.at[:, pl.ds(h * HALF, HALF)], tab_v)

    cols = [lax.iota(jnp.int32, L) + cj * L for cj in range(HALF // L)]

    def stage_add(i):
        x_desc(i).wait()
        t = tch(i)[1]
        xb = xbufs[i % 3]

        def add_step(r, _):
            d = plsc.load_gather(idx_v, [jnp.full((L,), t * CHUNK, jnp.int32) + r])
            for cj in range(HALF // L):
                sl = pl.ds(cj * L, L)
                tv = plsc.load_gather(tab_v, [d, cols[cj]])
                xb[r, sl] = xb[r, sl] + tv
            return 0

        lax.fori_loop(0, CHUNK, add_step, 0)
        pltpu.async_copy(xb, out_hbm.at[chunkid(i), :, xslice(i)], sem_o)

    dpa = pltpu.async_copy(
        partial_hbm.at[pl.ds(wid * P_WORDS, P_WORDS)], pa_v, sem_p)
    dpb = pltpu.async_copy(
        partial_hbm.at[pl.ds(HIST_PAD + wid * P_WORDS, P_WORDS)], pb_v, sem_p)
    pl.when(valid(0))(lambda: fire_x(0))
    pl.when(valid(1))(lambda: fire_x(1))
    zrows = MAX_DEG // NS
    pltpu.sync_copy(zsum_hbm.at[pl.ds(sid * zrows, zrows)],
                    zs_sh.at[pl.ds(sid * zrows, zrows)])
    plsc.subcore_barrier()
    load_table(0)
    dpa.wait()
    dpb.wait()
    for j in range(P_WORDS // L):
        sl = pl.ds(j * L, L)
        idx_v[sl] = jnp.minimum(pa_v[sl] + pb_v[sl], MAX_DEG - 1)

    for i in range(N_ITEMS):
        if i + 2 < N_ITEMS:
            if i - 1 >= 0:
                pl.when(valid(i - 1))(lambda i=i: o_desc(i - 1).wait())
            pl.when(valid(i + 2))(lambda i=i: fire_x(i + 2))
        pl.when(valid(i))(lambda i=i: stage_add(i))
        if i == MAX_T - 1:
            load_table(1)

    for i in range(N_ITEMS):
        if i + 3 >= N_ITEMS:
            pl.when(valid(i))(lambda i=i: o_desc(i).wait())


_gather_kernel = functools.partial(
    pl.kernel,
    out_type=jax.ShapeDtypeStruct((N_CHUNKS, CHUNK, NODE_DIM), jnp.float32),
    mesh=_MESH,
    compiler_params=pltpu.CompilerParams(needs_layout_passes=False),
    scratch_types=[
        pltpu.VMEM((P_WORDS,), jnp.int32),
        pltpu.VMEM((P_WORDS,), jnp.int32),
        pltpu.VMEM((P_WORDS,), jnp.int32),
        pltpu.VMEM((CHUNK, HALF), jnp.float32),
        pltpu.VMEM((CHUNK, HALF), jnp.float32),
        pltpu.VMEM((CHUNK, HALF), jnp.float32),
        pltpu.VMEM((MAX_DEG, HALF), jnp.float32),
        pltpu.VMEM_SHARED((MAX_DEG, NODE_DIM), jnp.float32),
        pltpu.SemaphoreType.DMA,
        pltpu.SemaphoreType.DMA((N_ITEMS,)),
        pltpu.SemaphoreType.DMA,
    ],
)(_gather_body)


def kernel(x, edge_index, z_in, z_out):
    edges = edge_index.reshape(-1)
    partial, zsum = _hist_kernel(edges, z_in, z_out)
    x3 = x.reshape(N_CHUNKS, CHUNK, NODE_DIM)
    out3 = _gather_kernel(x3, partial, zsum)
    return out3.reshape(N_NODES, NODE_DIM)


# speedup vs baseline: 1.6324x; 1.0937x over previous
import functools

import jax
import jax.numpy as jnp
from jax import lax
from jax.experimental import pallas as pl
from jax.experimental.pallas import tpu as pltpu
from jax.experimental.pallas import tpu_sc as plsc

N_NODES = 10000
N_EDGES = 160000
NODE_DIM = 256
MAX_DEG = 512

NC = 2
NS = 16
NW = NC * NS
L = 16

N_EP = 2 * N_EDGES
EP_PER_TILE = N_EP // NW
HIST_PAD = 10240
COLS_PER_TILE = HIST_PAD // NS
Z_ROWS_PER_TILE = MAX_DEG // NW

CHUNK = 80
N_CHUNKS = N_NODES // CHUNK
MAX_T = -(-N_CHUNKS // NW)

_MESH = plsc.VectorSubcoreMesh(
    core_axis_name="c", subcore_axis_name="s", num_cores=NC, num_subcores=NS
)


def _hist_body(edges_hbm, zin_hbm, zout_hbm, partial_hbm, zsum_hbm,
               ep_v, hist_v, colbuf_v, red_v, zi_v, zo_v, shared_hist,
               sem_e, sem_z, sem_zo):
    cid = lax.axis_index("c")
    sid = lax.axis_index("s")
    wid = cid * NS + sid

    cp_ep = pltpu.async_copy(
        edges_hbm.at[pl.ds(wid * EP_PER_TILE, EP_PER_TILE)], ep_v, sem_e)
    zrow = wid * Z_ROWS_PER_TILE
    cp_zi = pltpu.async_copy(zin_hbm.at[pl.ds(zrow, Z_ROWS_PER_TILE)], zi_v, sem_z)
    cp_zo = pltpu.async_copy(zout_hbm.at[pl.ds(zrow, Z_ROWS_PER_TILE)], zo_v, sem_z)

    zeros = jnp.zeros((L,), jnp.int32)

    def zero_step(j, _):
        for u in range(4):
            hist_v[pl.ds((j * 4 + u) * L, L)] = zeros
        return 0

    lax.fori_loop(0, HIST_PAD // L // 4, zero_step, 0)

    cp_zi.wait()
    cp_zo.wait()

    def zsum_step(r, _):
        for cj in range(NODE_DIM // L):
            sl = pl.ds(cj * L, L)
            zi_v[r, sl] = zi_v[r, sl] + zo_v[r, sl]
        return 0

    lax.fori_loop(0, Z_ROWS_PER_TILE, zsum_step, 0)
    cp_zs = pltpu.async_copy(
        zi_v, zsum_hbm.at[pl.ds(zrow, Z_ROWS_PER_TILE)], sem_zo)

    cp_ep.wait()

    def hist_step(j, _):
        for u in range(5):
            v = ep_v[pl.ds((j * 5 + u) * L, L)]
            cnt, last = plsc.scan_count(v)
            plsc.addupdate_scatter(hist_v, [v], cnt, mask=last)
        return 0

    lax.fori_loop(0, EP_PER_TILE // L // 5, hist_step, 0)

    pltpu.sync_copy(hist_v, shared_hist.at[sid])
    plsc.subcore_barrier()
    pltpu.sync_copy(shared_hist.at[:, pl.ds(sid * COLS_PER_TILE, COLS_PER_TILE)],
                    colbuf_v)

    def red_step(j, _):
        acc = colbuf_v[0, pl.ds(j * L, L)]
        for r in range(1, NS):
            acc = acc + colbuf_v[r, pl.ds(j * L, L)]
        red_v[pl.ds(j * L, L)] = acc
        return 0

    lax.fori_loop(0, COLS_PER_TILE // L, red_step, 0)

    pltpu.sync_copy(
        red_v,
        partial_hbm.at[pl.ds(cid * HIST_PAD + sid * COLS_PER_TILE,
                             COLS_PER_TILE)],
    )
    cp_zs.wait()


_hist_kernel = functools.partial(
    pl.kernel,
    out_type=(
        jax.ShapeDtypeStruct((NC * HIST_PAD,), jnp.int32),
        jax.ShapeDtypeStruct((MAX_DEG, NODE_DIM), jnp.float32),
    ),
    mesh=_MESH,
    compiler_params=pltpu.CompilerParams(needs_layout_passes=False),
    scratch_types=[
        pltpu.VMEM((EP_PER_TILE,), jnp.int32),
        pltpu.VMEM((HIST_PAD,), jnp.int32),
        pltpu.VMEM((NS, COLS_PER_TILE), jnp.int32),
        pltpu.VMEM((COLS_PER_TILE,), jnp.int32),
        pltpu.VMEM((Z_ROWS_PER_TILE, NODE_DIM), jnp.float32),
        pltpu.VMEM((Z_ROWS_PER_TILE, NODE_DIM), jnp.float32),
        pltpu.VMEM_SHARED((NS, HIST_PAD), jnp.int32),
        pltpu.SemaphoreType.DMA,
        pltpu.SemaphoreType.DMA,
        pltpu.SemaphoreType.DMA,
    ],
)(_hist_body)


P_WORDS = MAX_T * CHUNK


HALF = NODE_DIM // 2
N_ITEMS = 2 * MAX_T


def _gather_body(x_hbm, partial_hbm, zsum_hbm, out_hbm,
                 pa_v, pb_v, idx_v, x0_v, x1_v, x2_v, tab_v, zs_sh,
                 sem_p, sem_x, sem_o):
    cid = lax.axis_index("c")
    sid = lax.axis_index("s")
    wid = cid * NS + sid

    xbufs = [x0_v, x1_v, x2_v]

    def tch(i):
        return i // MAX_T, i % MAX_T

    def valid(i):
        return (MAX_T * wid + tch(i)[1]) < N_CHUNKS

    def chunkid(i):
        return MAX_T * wid + tch(i)[1]

    def xslice(i):
        h = tch(i)[0]
        return pl.ds(h * HALF, HALF)

    def x_desc(i):
        return pltpu.make_async_copy(
            x_hbm.at[chunkid(i), :, xslice(i)], xbufs[i % 3], sem_x.at[i])

    def o_desc(i):
        return pltpu.make_async_copy(
            xbufs[i % 3], out_hbm.at[chunkid(i), :, xslice(i)], sem_o)

    def fire_x(i):
        pltpu.async_copy(
            x_hbm.at[chunkid(i), :, xslice(i)], xbufs[i % 3], sem_x.at[i])

    def load_table(h):
        pltpu.sync_copy(zs_sh.at[:, pl.ds(h * HALF, HALF)], tab_v)

    cols = [lax.iota(jnp.int32, L) + cj * L for cj in range(HALF // L)]
    lanes = [jnp.full((L, 1), r, jnp.int32) for r in range(L)]
    _dnums = lax.GatherDimensionNumbers(
        offset_dims=(), collapsed_slice_dims=(0,), start_index_map=(0,))

    def _bcast_lane(vec, r):
        return lax.gather(vec, lanes[r], dimension_numbers=_dnums,
                          slice_sizes=(1,),
                          mode=lax.GatherScatterMode.PROMISE_IN_BOUNDS)

    def stage_add(i):
        x_desc(i).wait()
        t = tch(i)[1]
        xb = xbufs[i % 3]

        def add_step(g, _):
            dvec = idx_v[pl.ds(t * CHUNK + g * L, L)]
            for r in range(L):
                dr = _bcast_lane(dvec, r)
                row = g * L + r
                for cj in range(HALF // L):
                    tv = plsc.load_gather(tab_v, [dr, cols[cj]])
                    plsc.addupdate(xb.at[row, pl.ds(cj * L, L)], tv)
            return 0

        lax.fori_loop(0, CHUNK // L, add_step, 0)
        pltpu.async_copy(xb, out_hbm.at[chunkid(i), :, xslice(i)], sem_o)

    dpa = pltpu.async_copy(
        partial_hbm.at[pl.ds(wid * P_WORDS, P_WORDS)], pa_v, sem_p)
    dpb = pltpu.async_copy(
        partial_hbm.at[pl.ds(HIST_PAD + wid * P_WORDS, P_WORDS)], pb_v, sem_p)
    pl.when(valid(0))(lambda: fire_x(0))
    pl.when(valid(1))(lambda: fire_x(1))
    zrows = MAX_DEG // NS
    pltpu.sync_copy(zsum_hbm.at[pl.ds(sid * zrows, zrows)],
                    zs_sh.at[pl.ds(sid * zrows, zrows)])
    plsc.subcore_barrier()
    load_table(0)
    dpa.wait()
    dpb.wait()
    for j in range(P_WORDS // L):
        sl = pl.ds(j * L, L)
        idx_v[sl] = jnp.minimum(pa_v[sl] + pb_v[sl], MAX_DEG - 1)

    for i in range(N_ITEMS):
        if i + 2 < N_ITEMS:
            if i - 1 >= 0:
                pl.when(valid(i - 1))(lambda i=i: o_desc(i - 1).wait())
            pl.when(valid(i + 2))(lambda i=i: fire_x(i + 2))
        pl.when(valid(i))(lambda i=i: stage_add(i))
        if i == MAX_T - 1:
            load_table(1)

    for i in range(N_ITEMS):
        if i + 3 >= N_ITEMS:
            pl.when(valid(i))(lambda i=i: o_desc(i).wait())


_gather_kernel = functools.partial(
    pl.kernel,
    out_type=jax.ShapeDtypeStruct((N_CHUNKS, CHUNK, NODE_DIM), jnp.float32),
    mesh=_MESH,
    compiler_params=pltpu.CompilerParams(needs_layout_passes=False),
    scratch_types=[
        pltpu.VMEM((P_WORDS,), jnp.int32),
        pltpu.VMEM((P_WORDS,), jnp.int32),
        pltpu.VMEM((P_WORDS,), jnp.int32),
        pltpu.VMEM((CHUNK, HALF), jnp.float32),
        pltpu.VMEM((CHUNK, HALF), jnp.float32),
        pltpu.VMEM((CHUNK, HALF), jnp.float32),
        pltpu.VMEM((MAX_DEG, HALF), jnp.float32),
        pltpu.VMEM_SHARED((MAX_DEG, NODE_DIM), jnp.float32),
        pltpu.SemaphoreType.DMA,
        pltpu.SemaphoreType.DMA((N_ITEMS,)),
        pltpu.SemaphoreType.DMA,
    ],
)(_gather_body)


def kernel(x, edge_index, z_in, z_out):
    edges = edge_index.reshape(-1)
    partial, zsum = _hist_kernel(edges, z_in, z_out)
    x3 = x.reshape(N_CHUNKS, CHUNK, NODE_DIM)
    out3 = _gather_kernel(x3, partial, zsum)
    return out3.reshape(N_NODES, NODE_DIM)
